# Initial kernel scaffold; baseline (speedup 1.0000x reference)
import jax
import jax.numpy as jnp
from jax.experimental import pallas as pl


def kernel(x, W1, b1, g1, be1, W2, b2, Wg, bg):
    raise NotImplementedError("write your pallas kernel here")


# speedup vs baseline: 3.1644x; 3.1644x over previous
import functools
import math

import jax
import jax.numpy as jnp
from jax.experimental import pallas as pl
from jax.experimental.pallas import tpu as pltpu

N = 2048
D = 1024
H = 1024
E = 8
K = 2
EPS = 1e-5
_INV_SQRT2 = 1.0 / math.sqrt(2.0)


def _moe_body(x_ref, w1_ref, b1_ref, g1_ref, be1_ref, w2_ref, b2_ref,
              wg_ref, bg_ref, out_ref, w_sc, xb_sc):
    e = pl.program_id(0)

    @pl.when(e == 0)
    def _gate():
        x = x_ref[...]
        xb_sc[...] = x.astype(jnp.bfloat16)
        logits = jnp.dot(x, wg_ref[...],
                         preferred_element_type=jnp.float32) + bg_ref[0]
        m = jnp.max(logits, axis=-1, keepdims=True)
        ex = jnp.exp(logits - m)
        probs = ex / jnp.sum(ex, axis=-1, keepdims=True)
        cols = jax.lax.broadcasted_iota(jnp.int32, (1, E), 1)
        masks = []
        for j in range(E):
            pj = probs[:, j:j + 1]
            rank = (jnp.sum((probs > pj).astype(jnp.float32), axis=1,
                            keepdims=True)
                    + jnp.sum(((probs == pj) & (cols < j)).astype(jnp.float32),
                              axis=1, keepdims=True))
            masks.append((rank < float(K)).astype(jnp.float32))
        mask = jnp.concatenate(masks, axis=1)
        wsel = probs * mask
        w_sc[...] = wsel / jnp.sum(wsel, axis=1, keepdims=True)
        out_ref[...] = jnp.zeros_like(out_ref)

    xb = xb_sc[...]
    h = jnp.dot(xb, w1_ref[0].astype(jnp.bfloat16),
                preferred_element_type=jnp.float32) + b1_ref[0]
    mu = jnp.mean(h, axis=-1, keepdims=True)
    var = jnp.mean((h - mu) * (h - mu), axis=-1, keepdims=True)
    hn = (h - mu) * jax.lax.rsqrt(var + EPS) * g1_ref[0] + be1_ref[0]
    ha = hn * 0.5 * (1.0 + jax.lax.erf(hn * _INV_SQRT2))
    eo = jnp.dot(ha.astype(jnp.bfloat16), w2_ref[0].astype(jnp.bfloat16),
                 preferred_element_type=jnp.float32) + b2_ref[0]
    ecols = jax.lax.broadcasted_iota(jnp.int32, (N, E), 1)
    scale = jnp.sum(jnp.where(ecols == e, w_sc[...], 0.0), axis=1,
                    keepdims=True)
    out_ref[...] += eo * scale


@jax.jit
def kernel(x, W1, b1, g1, be1, W2, b2, Wg, bg):
    bg2 = bg.reshape(1, E)
    b1r = b1.reshape(E, 1, H)
    g1r = g1.reshape(E, 1, H)
    be1r = be1.reshape(E, 1, H)
    b2r = b2.reshape(E, 1, H)
    grid = (E,)
    out = pl.pallas_call(
        _moe_body,
        grid=grid,
        in_specs=[
            pl.BlockSpec((N, D), lambda e: (0, 0)),
            pl.BlockSpec((1, D, H), lambda e: (e, 0, 0)),
            pl.BlockSpec((1, 1, H), lambda e: (e, 0, 0)),
            pl.BlockSpec((1, 1, H), lambda e: (e, 0, 0)),
            pl.BlockSpec((1, 1, H), lambda e: (e, 0, 0)),
            pl.BlockSpec((1, H, H), lambda e: (e, 0, 0)),
            pl.BlockSpec((1, 1, H), lambda e: (e, 0, 0)),
            pl.BlockSpec((D, E), lambda e: (0, 0)),
            pl.BlockSpec((1, E), lambda e: (0, 0)),
        ],
        out_specs=pl.BlockSpec((N, H), lambda e: (0, 0)),
        out_shape=jax.ShapeDtypeStruct((N, H), jnp.float32),
        scratch_shapes=[
            pltpu.VMEM((N, E), jnp.float32),
            pltpu.VMEM((N, D), jnp.bfloat16),
        ],
    )(x, W1, b1r, g1r, be1r, W2, b2r, Wg, bg2)
    return out
